# baseline (speedup 1.0000x reference)
import math

import jax
import jax.numpy as jnp
from jax.experimental import pallas as pl
from jax.experimental.pallas import tpu as pltpu

_B = 16
_N = 256
_G = _N // _B

_HI = 0.2
_LO = 0.1

_W0 = 0.09228515625
_W1 = 0.055908203125
_W2 = 0.033935546875
_W3 = 0.0206298828125


def _shift_h(a, d):
    z = jnp.zeros_like(a[:, :1, :])
    if d == 1:
        return jnp.concatenate([a[:, 1:, :], z], axis=1)
    return jnp.concatenate([z, a[:, :-1, :]], axis=1)


def _shift_w(a, d):
    z = jnp.zeros_like(a[:, :, :1])
    if d == 1:
        return jnp.concatenate([a[:, :, 1:], z], axis=2)
    return jnp.concatenate([z, a[:, :, :-1]], axis=2)


def _canny_kernel(a_ref, b_ref, o_ref):
    d0 = pl.program_id(0) * _B - 4

    w = jnp.concatenate([a_ref[...], b_ref[...]], axis=0)
    wb = w.astype(jnp.bfloat16).astype(jnp.float32)

    V = jnp.concatenate([wb[:, 1:2, :], wb, wb[:, -2:-1, :]], axis=1)
    V = jnp.concatenate([V[:, :, 1:2], V, V[:, :, -2:-1]], axis=2)

    Pd = lambda t: t[:-2] + t[2:]
    Ph = lambda t: t[:, :-2] + t[:, 2:]
    Pw = lambda t: t[:, :, :-2] + t[:, :, 2:]
    PdV = Pd(V)
    PhV = Ph(V)
    PhPdV = Ph(PdV)
    F = PdV[:, 1:-1, 1:-1] + PhV[1:-1, :, 1:-1] + Pw(V)[1:-1, 1:-1, :]
    E = PhPdV[:, :, 1:-1] + Pw(PdV)[:, 1:-1, :] + Pw(PhV)[1:-1, :, :]
    C8 = Pw(PhPdV)
    s = _W0 * V[1:-1, 1:-1, 1:-1] + _W1 * F + _W2 * E + _W3 * C8
    s = s.astype(jnp.bfloat16).astype(jnp.float32)

    A = s[:-2] + s[1:-1] + s[2:]
    C = s[2:] - s[:-2]
    As = A[:, :-2, :] + 2.0 * A[:, 1:-1, :] + A[:, 2:, :]
    Ad = A[:, 2:, :] - A[:, :-2, :]
    Cb = C[:, :-2, :] + C[:, 1:-1, :] + C[:, 2:, :]
    gx = As[:, :, 2:] - As[:, :, :-2]
    gy = Ad[:, :, :-2] + 2.0 * Ad[:, :, 1:-1] + Ad[:, :, 2:]
    gz = Cb[:, :, :-2] + Cb[:, :, 1:-1] + Cb[:, :, 2:]
    magc = jnp.sqrt(gx * gx + gy * gy + gz * gz)

    zh = jnp.zeros_like(magc[:, :1, :])
    mag = jnp.concatenate([zh, magc, zh], axis=1)
    zw = jnp.zeros_like(mag[:, :, :1])
    mag = jnp.concatenate([zw, mag, zw], axis=2)
    m_ids = jax.lax.broadcasted_iota(jnp.int32, mag.shape, 0) + (d0 + 2)
    mag = jnp.where((m_ids >= 1) & (m_ids <= _N - 2), mag, 0.0)

    cur = mag[1:_B + 3]
    prev = mag[:_B + 2]
    p_l = _shift_w(prev, -1)
    p_r = _shift_w(prev, 1)
    a3 = jnp.maximum(prev, jnp.maximum(p_l, p_r))
    m8 = jnp.maximum(
        jnp.maximum(_shift_h(a3, -1), _shift_h(a3, 1)),
        jnp.maximum(p_l, p_r))
    nms = jnp.where(cur > m8, cur, 0.0)

    strong = nms > _HI
    sf = strong.astype(jnp.float32)
    scf = sf[1:_B + 1]

    any6 = jnp.maximum(sf[:_B], sf[2:_B + 2])
    any6 = jnp.maximum(any6, jnp.maximum(_shift_h(scf, -1), _shift_h(scf, 1)))
    any6 = jnp.maximum(any6, jnp.maximum(_shift_w(scf, -1), _shift_w(scf, 1)))
    nc = nms[1:_B + 1]
    conn = strong[1:_B + 1] | ((nc > _LO) & (nc <= _HI) & (any6 > 0.0))
    o_ref[...] = conn.astype(jnp.int8)


def kernel(x):
    x3 = x[0]
    z3 = jnp.zeros((3, _N, _N), x3.dtype)
    xq = jnp.concatenate([z3, x3[1:2], x3, x3[_N - 2:_N - 1], z3], axis=0)

    out = pl.pallas_call(
        _canny_kernel,
        out_shape=jax.ShapeDtypeStruct((_N, _N, _N), jnp.int8),
        grid=(_G,),
        in_specs=[
            pl.BlockSpec((_B, _N, _N), lambda i: (i, 0, 0)),
            pl.BlockSpec((8, _N, _N), lambda i: ((i + 1) * (_B // 8), 0, 0)),
        ],
        out_specs=pl.BlockSpec((_B, _N, _N), lambda i: (i, 0, 0)),
        compiler_params=pltpu.CompilerParams(
            dimension_semantics=("parallel",),
            vmem_limit_bytes=60 * 1024 * 1024,
        ),
        name="canny3d",
    )(xq, xq)
    return out[None]


# speedup vs baseline: 1.0821x; 1.0821x over previous
import math

import jax
import jax.numpy as jnp
from jax.experimental import pallas as pl
from jax.experimental.pallas import tpu as pltpu

_B = 8
_N = 256
_G = _N // _B

_HI = 0.2
_LO = 0.1

_W0 = 0.09228515625
_W1 = 0.055908203125
_W2 = 0.033935546875
_W3 = 0.0206298828125


def _shift_h(a, d):
    z = jnp.zeros_like(a[:, :1, :])
    if d == 1:
        return jnp.concatenate([a[:, 1:, :], z], axis=1)
    return jnp.concatenate([z, a[:, :-1, :]], axis=1)


def _shift_w(a, d):
    z = jnp.zeros_like(a[:, :, :1])
    if d == 1:
        return jnp.concatenate([a[:, :, 1:], z], axis=2)
    return jnp.concatenate([z, a[:, :, :-1]], axis=2)


def _canny_kernel(a_ref, b_ref, o_ref):
    d0 = pl.program_id(0) * _B - 4

    w = jnp.concatenate([a_ref[...], b_ref[...]], axis=0)
    wb = w.astype(jnp.bfloat16).astype(jnp.float32)

    V = jnp.concatenate([wb[:, 1:2, :], wb, wb[:, -2:-1, :]], axis=1)
    V = jnp.concatenate([V[:, :, 1:2], V, V[:, :, -2:-1]], axis=2)

    Pd = lambda t: t[:-2] + t[2:]
    Ph = lambda t: t[:, :-2] + t[:, 2:]
    Pw = lambda t: t[:, :, :-2] + t[:, :, 2:]
    PdV = Pd(V)
    PhV = Ph(V)
    PhPdV = Ph(PdV)
    F = PdV[:, 1:-1, 1:-1] + PhV[1:-1, :, 1:-1] + Pw(V)[1:-1, 1:-1, :]
    E = PhPdV[:, :, 1:-1] + Pw(PdV)[:, 1:-1, :] + Pw(PhV)[1:-1, :, :]
    C8 = Pw(PhPdV)
    s = _W0 * V[1:-1, 1:-1, 1:-1] + _W1 * F + _W2 * E + _W3 * C8
    s = s.astype(jnp.bfloat16).astype(jnp.float32)

    A = s[:-2] + s[1:-1] + s[2:]
    C = s[2:] - s[:-2]
    As = A[:, :-2, :] + 2.0 * A[:, 1:-1, :] + A[:, 2:, :]
    Ad = A[:, 2:, :] - A[:, :-2, :]
    Cb = C[:, :-2, :] + C[:, 1:-1, :] + C[:, 2:, :]
    gx = As[:, :, 2:] - As[:, :, :-2]
    gy = Ad[:, :, :-2] + 2.0 * Ad[:, :, 1:-1] + Ad[:, :, 2:]
    gz = Cb[:, :, :-2] + Cb[:, :, 1:-1] + Cb[:, :, 2:]
    magc = jnp.sqrt(gx * gx + gy * gy + gz * gz)

    zh = jnp.zeros_like(magc[:, :1, :])
    mag = jnp.concatenate([zh, magc, zh], axis=1)
    zw = jnp.zeros_like(mag[:, :, :1])
    mag = jnp.concatenate([zw, mag, zw], axis=2)
    m_ids = jax.lax.broadcasted_iota(jnp.int32, mag.shape, 0) + (d0 + 2)
    mag = jnp.where((m_ids >= 1) & (m_ids <= _N - 2), mag, 0.0)

    cur = mag[1:_B + 3]
    prev = mag[:_B + 2]
    p_l = _shift_w(prev, -1)
    p_r = _shift_w(prev, 1)
    a3 = jnp.maximum(prev, jnp.maximum(p_l, p_r))
    m8 = jnp.maximum(
        jnp.maximum(_shift_h(a3, -1), _shift_h(a3, 1)),
        jnp.maximum(p_l, p_r))
    nms = jnp.where(cur > m8, cur, 0.0)

    strong = nms > _HI
    sf = strong.astype(jnp.float32)
    scf = sf[1:_B + 1]

    any6 = jnp.maximum(sf[:_B], sf[2:_B + 2])
    any6 = jnp.maximum(any6, jnp.maximum(_shift_h(scf, -1), _shift_h(scf, 1)))
    any6 = jnp.maximum(any6, jnp.maximum(_shift_w(scf, -1), _shift_w(scf, 1)))
    nc = nms[1:_B + 1]
    conn = strong[1:_B + 1] | ((nc > _LO) & (nc <= _HI) & (any6 > 0.0))
    o_ref[...] = conn.astype(jnp.int8)


def kernel(x):
    x3 = x[0]
    z3 = jnp.zeros((3, _N, _N), x3.dtype)
    xq = jnp.concatenate([z3, x3[1:2], x3, x3[_N - 2:_N - 1], z3], axis=0)

    out = pl.pallas_call(
        _canny_kernel,
        out_shape=jax.ShapeDtypeStruct((_N, _N, _N), jnp.int8),
        grid=(_G,),
        in_specs=[
            pl.BlockSpec((_B, _N, _N), lambda i: (i, 0, 0)),
            pl.BlockSpec((8, _N, _N), lambda i: ((i + 1) * (_B // 8), 0, 0)),
        ],
        out_specs=pl.BlockSpec((_B, _N, _N), lambda i: (i, 0, 0)),
        compiler_params=pltpu.CompilerParams(
            dimension_semantics=("parallel",),
            vmem_limit_bytes=56 * 1024 * 1024,
        ),
        name="canny3d",
    )(xq, xq)
    return out[None]
